# Initial kernel scaffold; baseline (speedup 1.0000x reference)
import jax
import jax.numpy as jnp
from jax.experimental import pallas as pl


def kernel(inputs):
    raise NotImplementedError("write your pallas kernel here")


# speedup vs baseline: 13.0697x; 13.0697x over previous
import jax
import jax.numpy as jnp
from jax.experimental import pallas as pl
from jax.experimental.pallas import tpu as pltpu

_N_ITERS = 30
_EPS = 1e-12


def _entmax_block_kernel(x_ref, o_ref):
    x = x_ref[...]
    rowmax = jnp.max(x, axis=1, keepdims=True)
    lo = rowmax - 1.0
    hi = rowmax

    def body(_, carry):
        lo, hi = carry
        mid = 0.5 * (lo + hi)
        u = jnp.maximum(x - mid, 0.0)
        f = jnp.sum(u * u, axis=1, keepdims=True)
        ge = f >= 1.0
        lo = jnp.where(ge, mid, lo)
        hi = jnp.where(ge, hi, mid)
        return lo, hi

    lo, hi = jax.lax.fori_loop(0, _N_ITERS, body, (lo, hi))
    tau = 0.5 * (lo + hi)
    y = jnp.maximum(x - tau, 0.0)
    y = y * y
    norm = jnp.maximum(jnp.sum(y, axis=1, keepdims=True), _EPS)
    o_ref[...] = y / norm


def kernel(inputs):
    rows, n = inputs.shape
    block_rows = 8
    grid = (rows // block_rows,)
    return pl.pallas_call(
        _entmax_block_kernel,
        grid=grid,
        in_specs=[pl.BlockSpec((block_rows, n), lambda i: (i, 0))],
        out_specs=pl.BlockSpec((block_rows, n), lambda i: (i, 0)),
        out_shape=jax.ShapeDtypeStruct((rows, n), inputs.dtype),
        compiler_params=pltpu.CompilerParams(
            dimension_semantics=("parallel",),
        ),
    )(inputs)


# speedup vs baseline: 35.2699x; 2.6986x over previous
import jax
import jax.numpy as jnp
from jax.experimental import pallas as pl
from jax.experimental.pallas import tpu as pltpu

_N_ITERS = 8
_EPS = 1e-12


def _entmax_block_kernel(x_ref, o_ref):
    x = x_ref[...]
    rowmax = jnp.max(x, axis=1, keepdims=True)
    lo = rowmax - 1.0
    hi = rowmax
    t = rowmax - 0.5

    def body(_, carry):
        lo, hi, t = carry
        u = jnp.maximum(x - t, 0.0)
        c = jnp.where(u > 0.0, 1.0, 0.0)
        k = jnp.sum(c, axis=1, keepdims=True)
        s1 = jnp.sum(u, axis=1, keepdims=True)
        s2 = jnp.sum(u * u, axis=1, keepdims=True)
        ge = s2 >= 1.0
        s1g = jnp.maximum(s1, 1e-30)
        tN = t + (s2 - 1.0) / (2.0 * s1g)
        kk = jnp.maximum(k, 1.0)
        disc = jnp.maximum(s1 * s1 - kk * (s2 - 1.0), 0.0)
        tc = t + (s1 - jnp.sqrt(disc)) / kk
        lo = jnp.maximum(lo, tN)
        lo = jnp.where(ge, lo, jnp.maximum(lo, tc))
        hi = jnp.where(ge, jnp.minimum(hi, tc), jnp.minimum(hi, t))
        t = 0.5 * (lo + hi)
        return lo, hi, t

    lo, hi, t = jax.lax.fori_loop(0, _N_ITERS, body, (lo, hi, t))
    tau = lo
    y = jnp.maximum(x - tau, 0.0)
    y = y * y
    norm = jnp.maximum(jnp.sum(y, axis=1, keepdims=True), _EPS)
    o_ref[...] = y / norm


def kernel(inputs):
    rows, n = inputs.shape
    block_rows = 8
    grid = (rows // block_rows,)
    return pl.pallas_call(
        _entmax_block_kernel,
        grid=grid,
        in_specs=[pl.BlockSpec((block_rows, n), lambda i: (i, 0))],
        out_specs=pl.BlockSpec((block_rows, n), lambda i: (i, 0)),
        out_shape=jax.ShapeDtypeStruct((rows, n), inputs.dtype),
        compiler_params=pltpu.CompilerParams(
            dimension_semantics=("parallel",),
        ),
    )(inputs)


# speedup vs baseline: 64.6772x; 1.8338x over previous
import jax
import jax.numpy as jnp
from jax.experimental import pallas as pl
from jax.experimental.pallas import tpu as pltpu

_N_PASSES = 7
_CHUNK = 1024
_EPS = 1e-12


def _entmax_block_kernel(x_ref, o_ref):
    rows, n = x_ref.shape
    nch = n // _CHUNK

    m = x_ref[:, 0:_CHUNK]
    for j in range(1, nch):
        m = jnp.maximum(m, x_ref[:, j * _CHUNK:(j + 1) * _CHUNK])
    rowmax = jnp.max(m, axis=1, keepdims=True)
    t0 = rowmax - 1.0

    zero_chunk = jnp.zeros((rows, _CHUNK), jnp.float32)

    def body(_, carry):
        t, _tau_prev, _norm_prev = carry
        tb = t + zero_chunk
        s1a = zero_chunk
        s2a = zero_chunk
        for j in range(nch):
            u = jnp.maximum(x_ref[:, j * _CHUNK:(j + 1) * _CHUNK] - tb, 0.0)
            s1a = s1a + u
            s2a = s2a + u * u
        s1 = jnp.sum(s1a, axis=1, keepdims=True)
        s2 = jnp.sum(s2a, axis=1, keepdims=True)
        t_next = t + (s2 - 1.0) / (2.0 * jnp.maximum(s1, 1e-30))
        return t_next, t, s2

    _, tau, norm = jax.lax.fori_loop(
        0, _N_PASSES, body, (t0, t0, jnp.ones_like(t0))
    )
    inv = 1.0 / jnp.maximum(norm, _EPS)
    taub = tau + zero_chunk
    invb = inv + zero_chunk
    for j in range(nch):
        u = jnp.maximum(x_ref[:, j * _CHUNK:(j + 1) * _CHUNK] - taub, 0.0)
        o_ref[:, j * _CHUNK:(j + 1) * _CHUNK] = u * u * invb


def kernel(inputs):
    rows, n = inputs.shape
    block_rows = 8
    grid = (rows // block_rows,)
    return pl.pallas_call(
        _entmax_block_kernel,
        grid=grid,
        in_specs=[pl.BlockSpec((block_rows, n), lambda i: (i, 0))],
        out_specs=pl.BlockSpec((block_rows, n), lambda i: (i, 0)),
        out_shape=jax.ShapeDtypeStruct((rows, n), inputs.dtype),
        compiler_params=pltpu.CompilerParams(
            dimension_semantics=("parallel",),
        ),
    )(inputs)


# speedup vs baseline: 71.6937x; 1.1085x over previous
import jax
import jax.numpy as jnp
from jax.experimental import pallas as pl
from jax.experimental.pallas import tpu as pltpu

_N_PASSES = 6
_CHUNK = 512
_EPS = 1e-12


def _entmax_block_kernel(x_ref, o_ref):
    rows, n = x_ref.shape
    nch = n // _CHUNK

    m = x_ref[:, 0:_CHUNK]
    for j in range(1, nch):
        m = jnp.maximum(m, x_ref[:, j * _CHUNK:(j + 1) * _CHUNK])
    rowmax = jnp.max(m, axis=1, keepdims=True)
    t0 = rowmax - 1.0

    zero_chunk = jnp.zeros((rows, _CHUNK), jnp.float32)

    def body(_, carry):
        t, _tau_prev, _norm_prev = carry
        tb = t + zero_chunk
        def chunk_body(j, accs):
            s1a, s2a = accs
            u = jnp.maximum(x_ref[:, pl.ds(j * _CHUNK, _CHUNK)] - tb, 0.0)
            return s1a + u, s2a + u * u

        s1a, s2a = jax.lax.fori_loop(
            0, nch, chunk_body, (zero_chunk, zero_chunk), unroll=2
        )
        s1 = jnp.sum(s1a, axis=1, keepdims=True)
        s2 = jnp.sum(s2a, axis=1, keepdims=True)
        t_next = t + (s2 - 1.0) / (2.0 * jnp.maximum(s1, 1e-30))
        return t_next, t, s2

    _, tau, norm = jax.lax.fori_loop(
        0, _N_PASSES, body, (t0, t0, jnp.ones_like(t0))
    )
    inv = 1.0 / jnp.maximum(norm, _EPS)
    taub = tau + zero_chunk
    invb = inv + zero_chunk
    for j in range(nch):
        u = jnp.maximum(x_ref[:, j * _CHUNK:(j + 1) * _CHUNK] - taub, 0.0)
        o_ref[:, j * _CHUNK:(j + 1) * _CHUNK] = u * u * invb


def kernel(inputs):
    rows, n = inputs.shape
    block_rows = 32
    grid = (rows // block_rows,)
    return pl.pallas_call(
        _entmax_block_kernel,
        grid=grid,
        in_specs=[pl.BlockSpec((block_rows, n), lambda i: (i, 0))],
        out_specs=pl.BlockSpec((block_rows, n), lambda i: (i, 0)),
        out_shape=jax.ShapeDtypeStruct((rows, n), inputs.dtype),
        compiler_params=pltpu.CompilerParams(
            dimension_semantics=("parallel",),
        ),
    )(inputs)


# speedup vs baseline: 92.1190x; 1.2849x over previous
import jax
import jax.numpy as jnp
from jax.experimental import pallas as pl
from jax.experimental.pallas import tpu as pltpu

_N_PASSES = 6
_CHUNK = 512
_EPS = 1e-12


def _entmax_block_kernel(x_ref, o_ref):
    rows, n = x_ref.shape
    nch = n // _CHUNK

    m = x_ref[:, 0:_CHUNK]
    for j in range(1, nch):
        m = jnp.maximum(m, x_ref[:, j * _CHUNK:(j + 1) * _CHUNK])
    rowmax = jnp.max(m, axis=1, keepdims=True)
    t0 = rowmax - 1.0

    zero_chunk = jnp.zeros((rows, _CHUNK), jnp.float32)

    def body(_, carry):
        t, _tau_prev, _norm_prev = carry
        tb = t + zero_chunk
        s1a = jnp.zeros((rows, 128), jnp.float32)
        s2a = jnp.zeros((rows, 128), jnp.float32)
        for j in range(nch):
            u = jnp.maximum(x_ref[:, j * _CHUNK:(j + 1) * _CHUNK] - tb, 0.0)
            uu = u * u
            w = _CHUNK
            while w > 128:
                h = w // 2
                u = u[:, :h] + u[:, h:w]
                uu = uu[:, :h] + uu[:, h:w]
                w = h
            s1a = s1a + u
            s2a = s2a + uu
        s1 = jnp.sum(s1a, axis=1, keepdims=True)
        s2 = jnp.sum(s2a, axis=1, keepdims=True)
        t_next = t + (s2 - 1.0) / (2.0 * jnp.maximum(s1, 1e-30))
        return t_next, t, s2

    _, tau, norm = jax.lax.fori_loop(
        0, _N_PASSES, body, (t0, t0, jnp.ones_like(t0))
    )
    inv = 1.0 / jnp.maximum(norm, _EPS)
    taub = tau + zero_chunk
    invb = inv + zero_chunk
    for j in range(nch):
        u = jnp.maximum(x_ref[:, j * _CHUNK:(j + 1) * _CHUNK] - taub, 0.0)
        o_ref[:, j * _CHUNK:(j + 1) * _CHUNK] = u * u * invb


def kernel(inputs):
    rows, n = inputs.shape
    block_rows = 32
    grid = (rows // block_rows,)
    return pl.pallas_call(
        _entmax_block_kernel,
        grid=grid,
        in_specs=[pl.BlockSpec((block_rows, n), lambda i: (i, 0))],
        out_specs=pl.BlockSpec((block_rows, n), lambda i: (i, 0)),
        out_shape=jax.ShapeDtypeStruct((rows, n), inputs.dtype),
        compiler_params=pltpu.CompilerParams(
            dimension_semantics=("parallel",),
        ),
    )(inputs)
